# Initial kernel scaffold; baseline (speedup 1.0000x reference)
import jax
import jax.numpy as jnp
from jax.experimental import pallas as pl


def kernel(x, edge_index, edge_attr, W1, b1, g1, be1, m1, v1, W2, b2, g2, be2, m2, v2):
    raise NotImplementedError("write your pallas kernel here")


# speedup vs baseline: 7.5712x; 7.5712x over previous
import functools

import jax
import jax.numpy as jnp
from jax import lax
from jax.experimental import pallas as pl
from jax.experimental.pallas import tpu as pltpu
from jax.experimental.pallas import tpu_sc as plsc

N = 10000
NPAD = 10240
D = 128
HALF = 64
E = 320000
BN_EPS = 1e-5

NC = 2
NS = 16
CHUNK = 128
RPT = NPAD // NS

NCHUNK_A = 157
EPT_A = NCHUNK_A * CHUNK
EPAD_A = EPT_A * NS
NCHUNK_D = 158
EPAD_D = NCHUNK_D * CHUNK * NS
NODE_HALF = NPAD // 2
TRASH = NODE_HALF

_MESH = plsc.VectorSubcoreMesh(
    core_axis_name="c", subcore_axis_name="s", num_cores=NC, num_subcores=NS)
_SC_PARAMS = pltpu.CompilerParams(
    use_tc_tiling_on_sc=False, needs_layout_passes=False)


@functools.partial(
    pl.kernel,
    out_type=jax.ShapeDtypeStruct((NPAD, 16), jnp.float32),
    mesh=_MESH,
    scratch_types=[
        pltpu.VMEM((CHUNK,), jnp.int32),
        pltpu.VMEM((CHUNK,), jnp.float32),
        pltpu.VMEM((CHUNK, 16), jnp.float32),
        pltpu.VMEM((NODE_HALF // NS, 16), jnp.float32),
        pltpu.VMEM_SHARED((NODE_HALF + 16, 16), jnp.float32),
    ],
    compiler_params=_SC_PARAMS,
)
def _deg_kernel(col_hbm, w_hbm, out_hbm, ci_v, w_v, pay_v, buf_v, deg_sh):
    cid = lax.axis_index("c")
    sid = lax.axis_index("s")
    rows = NODE_HALF // NS
    lo = cid * NODE_HALF
    for i in range(rows):
        buf_v[i, :] = jnp.zeros((16,), jnp.float32)
    for i in range(CHUNK):
        pay_v[i, :] = jnp.zeros((16,), jnp.float32)
    pltpu.sync_copy(buf_v, deg_sh.at[pl.ds(sid * rows, rows)])
    plsc.subcore_barrier()

    ebase = sid * (NCHUNK_D * CHUNK)
    lanes = lax.iota(jnp.int32, 16)
    zeros16 = jnp.zeros((16,), jnp.int32)

    def chunk_body(i, _):
        base = ebase + i * CHUNK
        pltpu.sync_copy(col_hbm.at[pl.ds(base, CHUNK)], ci_v)
        pltpu.sync_copy(w_hbm.at[pl.ds(base, CHUNK)], w_v)
        for j in range(CHUNK // 16):
            c = ci_v[pl.ds(j * 16, 16)]
            cl = c - lo
            ok = (cl >= 0) & (cl < NODE_HALF)
            ci_v[pl.ds(j * 16, 16)] = jnp.where(ok, cl, TRASH)
            w16 = w_v[pl.ds(j * 16, 16)]
            plsc.store_scatter(pay_v, [lanes + j * 16, zeros16], w16)
        pltpu.sync_copy(pay_v, deg_sh.at[ci_v], add=True)
        return 0

    lax.fori_loop(0, NCHUNK_D, chunk_body, 0)
    plsc.subcore_barrier()
    pltpu.sync_copy(deg_sh.at[pl.ds(sid * rows, rows)], buf_v)
    pltpu.sync_copy(buf_v, out_hbm.at[pl.ds(lo + sid * rows, rows)])


@functools.partial(
    pl.kernel,
    out_type=jax.ShapeDtypeStruct((NC, NPAD, HALF), jnp.float32),
    mesh=_MESH,
    scratch_types=[
        pltpu.VMEM((CHUNK,), jnp.int32),
        pltpu.VMEM((CHUNK,), jnp.int32),
        pltpu.VMEM((CHUNK,), jnp.float32),
        pltpu.VMEM((CHUNK, HALF), jnp.float32),
        pltpu.VMEM((RPT, HALF), jnp.float32),
        pltpu.VMEM_SHARED((NPAD, HALF), jnp.float32),
        pltpu.SemaphoreType.DMA,
    ],
    compiler_params=_SC_PARAMS,
)
def _agg_kernel(xw_hbm, row_hbm, col_hbm, w_hbm, out_hbm,
                ri_v, ci_v, w_v, g_v, stage_v, acc_sh, sem):
    cid = lax.axis_index("c")
    sid = lax.axis_index("s")
    r0 = sid * RPT
    pltpu.sync_copy(xw_hbm.at[cid, pl.ds(r0, RPT)], stage_v)
    pltpu.sync_copy(stage_v, acc_sh.at[pl.ds(r0, RPT)])
    plsc.subcore_barrier()

    ebase = sid * EPT_A

    def chunk_body(i, _):
        base = ebase + i * CHUNK
        pltpu.sync_copy(row_hbm.at[pl.ds(base, CHUNK)], ri_v)
        pltpu.sync_copy(col_hbm.at[pl.ds(base, CHUNK)], ci_v)
        pltpu.sync_copy(w_hbm.at[pl.ds(base, CHUNK)], w_v)
        pltpu.async_copy(xw_hbm.at[cid].at[ri_v], g_v, sem).wait()

        def group_body(g, _):
            w16 = w_v[pl.ds(g * 16, 16)]
            for j in range(16):
                e = g * 16 + j
                wv = jnp.full((16,), w16[j])
                for k in range(HALF // 16):
                    g_v[e, pl.ds(k * 16, 16)] = g_v[e, pl.ds(k * 16, 16)] * wv
            return 0

        lax.fori_loop(0, CHUNK // 16, group_body, 0, unroll=2)
        pltpu.sync_copy(g_v, acc_sh.at[ci_v], add=True)
        return 0

    lax.fori_loop(0, NCHUNK_A, chunk_body, 0)
    plsc.subcore_barrier()
    pltpu.sync_copy(acc_sh.at[pl.ds(r0, RPT)], stage_v)
    pltpu.sync_copy(stage_v, out_hbm.at[cid, pl.ds(r0, RPT)])


def _tc_pre_body(deg_ref, x_ref, w_ref, dis_ref, xw_ref):
    dis = lax.rsqrt(deg_ref[:, 0:1] + 1.0)
    dis_ref[...] = dis
    xw = jnp.dot(x_ref[...], w_ref[...],
                 preferred_element_type=jnp.float32) * dis
    xw_ref[0] = xw[:, :HALF]
    xw_ref[1] = xw[:, HALF:]


def _tc_mid_body(acc_ref, dis_ref, b_ref, w_ref, xw_ref):
    dis = dis_ref[...]
    acc = jnp.concatenate([acc_ref[0], acc_ref[1]], axis=1)
    h = jnp.maximum(acc * dis + b_ref[...], 0.0)
    xw = jnp.dot(h, w_ref[...],
                 preferred_element_type=jnp.float32) * dis
    xw_ref[0] = xw[:, :HALF]
    xw_ref[1] = xw[:, HALF:]


def _tc_post_body(acc_ref, dis_ref, b_ref, out_ref):
    acc = jnp.concatenate([acc_ref[0], acc_ref[1]], axis=1)
    out_ref[...] = jnp.maximum(acc * dis_ref[...] + b_ref[...], 0.0)


_tc_pre = pl.pallas_call(
    _tc_pre_body,
    out_shape=[jax.ShapeDtypeStruct((NPAD, 1), jnp.float32),
               jax.ShapeDtypeStruct((NC, NPAD, HALF), jnp.float32)],
)

_tc_mid = pl.pallas_call(
    _tc_mid_body,
    out_shape=jax.ShapeDtypeStruct((NC, NPAD, HALF), jnp.float32),
)

_tc_post = pl.pallas_call(
    _tc_post_body,
    out_shape=jax.ShapeDtypeStruct((NPAD, D), jnp.float32),
)


def _fold_bn(W, b, g, be, m, v):
    s = g * lax.rsqrt(v + BN_EPS)
    return W * s[None, :], ((b - m) * s + be)[None, :]


def kernel(x, edge_index, edge_attr, W1, b1, g1, be1, m1, v1,
           W2, b2, g2, be2, m2, v2):
    row = edge_index[0]
    col = edge_index[1]
    zpad_i = jnp.zeros((EPAD_D - E,), jnp.int32)
    zpad_f = jnp.zeros((EPAD_D - E,), jnp.float32)
    rowp = jnp.concatenate([row, zpad_i])
    colp = jnp.concatenate([col, zpad_i])
    wp = jnp.concatenate([edge_attr, zpad_f])
    xp = jnp.pad(x, ((0, NPAD - N), (0, 0)))
    W1e, b1e = _fold_bn(W1, b1, g1, be1, m1, v1)
    W2e, b2e = _fold_bn(W2, b2, g2, be2, m2, v2)

    deg = _deg_kernel(colp, wp)
    dis, xw1 = _tc_pre(deg, xp, W1e)
    acc1 = _agg_kernel(xw1, rowp, colp, wp)
    xw2 = _tc_mid(acc1, dis, b1e, W2e)
    acc2 = _agg_kernel(xw2, rowp, colp, wp)
    out = _tc_post(acc2, dis, b2e)
    return out[:N]


# speedup vs baseline: 9.8694x; 1.3035x over previous
import functools

import jax
import jax.numpy as jnp
from jax import lax
from jax.experimental import pallas as pl
from jax.experimental.pallas import tpu as pltpu
from jax.experimental.pallas import tpu_sc as plsc

N = 10000
NPAD = 10240
D = 128
HALF = 64
E = 320000
BN_EPS = 1e-5

NC = 2
NS = 16
CHUNK = 128
RPT = NPAD // NS

SB = 4
NSUP = 40
NCHUNK = NSUP * SB
EPT = NCHUNK * CHUNK
EPAD = EPT * NS
SUP = SB * CHUNK
NODE_HALF = NPAD // 2
TRASH = NODE_HALF

_MESH = plsc.VectorSubcoreMesh(
    core_axis_name="c", subcore_axis_name="s", num_cores=NC, num_subcores=NS)
_SC_PARAMS = pltpu.CompilerParams(
    use_tc_tiling_on_sc=False, needs_layout_passes=False)


@functools.partial(
    pl.kernel,
    out_type=jax.ShapeDtypeStruct((NPAD, 16), jnp.float32),
    mesh=_MESH,
    scratch_types=[
        pltpu.VMEM((CHUNK,), jnp.int32),
        pltpu.VMEM((CHUNK,), jnp.float32),
        pltpu.VMEM((CHUNK, 16), jnp.float32),
        pltpu.VMEM((NODE_HALF // NS, 16), jnp.float32),
        pltpu.VMEM_SHARED((NODE_HALF + 16, 16), jnp.float32),
    ],
    compiler_params=_SC_PARAMS,
)
def _deg_kernel(col_hbm, w_hbm, out_hbm, ci_v, w_v, pay_v, buf_v, deg_sh):
    cid = lax.axis_index("c")
    sid = lax.axis_index("s")
    rows = NODE_HALF // NS
    lo = cid * NODE_HALF
    for i in range(rows):
        buf_v[i, :] = jnp.zeros((16,), jnp.float32)
    for i in range(CHUNK):
        pay_v[i, :] = jnp.zeros((16,), jnp.float32)
    pltpu.sync_copy(buf_v, deg_sh.at[pl.ds(sid * rows, rows)])
    plsc.subcore_barrier()

    ebase = sid * EPT
    lanes = lax.iota(jnp.int32, 16)
    zeros16 = jnp.zeros((16,), jnp.int32)

    def chunk_body(i, _):
        base = ebase + i * CHUNK
        pltpu.sync_copy(col_hbm.at[pl.ds(base, CHUNK)], ci_v)
        pltpu.sync_copy(w_hbm.at[pl.ds(base, CHUNK)], w_v)
        for j in range(CHUNK // 16):
            c = ci_v[pl.ds(j * 16, 16)]
            cl = c - lo
            ok = (cl >= 0) & (cl < NODE_HALF)
            ci_v[pl.ds(j * 16, 16)] = jnp.where(ok, cl, TRASH)
            w16 = w_v[pl.ds(j * 16, 16)]
            plsc.store_scatter(pay_v, [lanes + j * 16, zeros16], w16)
        pltpu.sync_copy(pay_v, deg_sh.at[ci_v], add=True)
        return 0

    lax.fori_loop(0, NCHUNK, chunk_body, 0)
    plsc.subcore_barrier()
    pltpu.sync_copy(deg_sh.at[pl.ds(sid * rows, rows)], buf_v)
    pltpu.sync_copy(buf_v, out_hbm.at[pl.ds(lo + sid * rows, rows)])


@functools.partial(
    pl.kernel,
    out_type=jax.ShapeDtypeStruct((NC, NPAD, HALF), jnp.float32),
    mesh=_MESH,
    scratch_types=[
        pltpu.VMEM((2, SB, 3, CHUNK), jnp.int32),
        pltpu.VMEM((2, SUP, HALF), jnp.float32),
        pltpu.VMEM_SHARED((NPAD, HALF), jnp.float32),
        pltpu.SemaphoreType.DMA,
        pltpu.SemaphoreType.DMA,
        pltpu.SemaphoreType.DMA,
        pltpu.SemaphoreType.DMA,
    ],
    compiler_params=_SC_PARAMS,
)
def _agg_kernel(xw_hbm, ed_hbm, out_hbm,
                ed_v, g_v, acc_sh, sg0, sg1, ss0, ss1):
    cid = lax.axis_index("c")
    sid = lax.axis_index("s")
    r0 = sid * RPT
    pltpu.sync_copy(xw_hbm.at[cid, pl.ds(r0, RPT)], acc_sh.at[pl.ds(r0, RPT)])
    plsc.subcore_barrier()

    cbase = sid * NCHUNK
    sgs = (sg0, sg1)
    sss = (ss0, ss1)

    def load_super(s, p):
        pltpu.sync_copy(ed_hbm.at[pl.ds(cbase + s * SB, SB)], ed_v.at[p])

    def fire_gathers(p):
        for b in range(SB):
            pltpu.async_copy(xw_hbm.at[cid].at[ed_v.at[p, b, 0]],
                             g_v.at[p, pl.ds(b * CHUNK, CHUNK)], sgs[p])

    def drain_gathers(p):
        for b in range(SB):
            pltpu.make_async_copy(xw_hbm.at[cid].at[ed_v.at[p, b, 0]],
                                  g_v.at[p, pl.ds(b * CHUNK, CHUNK)],
                                  sgs[p]).wait()

    def fire_scatters(p):
        for b in range(SB):
            pltpu.async_copy(g_v.at[p, pl.ds(b * CHUNK, CHUNK)],
                             acc_sh.at[ed_v.at[p, b, 1]], sss[p], add=True)

    def drain_scatters(p):
        for b in range(SB):
            pltpu.make_async_copy(g_v.at[p, pl.ds(b * CHUNK, CHUNK)],
                                  acc_sh.at[ed_v.at[p, b, 1]],
                                  sss[p]).wait()

    def scale(p):
        def group_body(g, _):
            b = lax.shift_right_logical(g, 3)
            go = lax.rem(g, 8)
            w16 = plsc.bitcast(ed_v[p, b, 2, pl.ds(go * 16, 16)], jnp.float32)
            for j in range(16):
                e = g * 16 + j
                wv = jnp.full((16,), w16[j])
                for k in range(HALF // 16):
                    g_v[p, e, pl.ds(k * 16, 16)] = (
                        g_v[p, e, pl.ds(k * 16, 16)] * wv)
            return 0
        lax.fori_loop(0, SUP // 16, group_body, 0)

    load_super(0, 0)
    fire_gathers(0)

    def pair_body(i, _):
        s0 = 2 * i
        load_super(s0 + 1, 1)
        fire_gathers(1)
        drain_gathers(0)
        scale(0)
        fire_scatters(0)
        drain_gathers(1)
        scale(1)
        fire_scatters(1)
        drain_scatters(0)
        drain_scatters(1)

        @pl.when(i < NSUP // 2 - 1)
        def _():
            load_super(s0 + 2, 0)
            fire_gathers(0)
        return 0

    lax.fori_loop(0, NSUP // 2, pair_body, 0)
    plsc.subcore_barrier()
    pltpu.sync_copy(acc_sh.at[pl.ds(r0, RPT)], out_hbm.at[cid, pl.ds(r0, RPT)])


def _tc_pre_body(deg_ref, x_ref, w_ref, dis_ref, xw_ref):
    dis = lax.rsqrt(deg_ref[:, 0:1] + 1.0)
    dis_ref[...] = dis
    xw = jnp.dot(x_ref[...], w_ref[...],
                 preferred_element_type=jnp.float32) * dis
    xw_ref[0] = xw[:, :HALF]
    xw_ref[1] = xw[:, HALF:]


def _tc_mid_body(acc_ref, dis_ref, b_ref, w_ref, xw_ref):
    dis = dis_ref[...]
    acc = jnp.concatenate([acc_ref[0], acc_ref[1]], axis=1)
    h = jnp.maximum(acc * dis + b_ref[...], 0.0)
    xw = jnp.dot(h, w_ref[...],
                 preferred_element_type=jnp.float32) * dis
    xw_ref[0] = xw[:, :HALF]
    xw_ref[1] = xw[:, HALF:]


def _tc_post_body(acc_ref, dis_ref, b_ref, out_ref):
    acc = jnp.concatenate([acc_ref[0], acc_ref[1]], axis=1)
    out_ref[...] = jnp.maximum(acc * dis_ref[...] + b_ref[...], 0.0)


_tc_pre = pl.pallas_call(
    _tc_pre_body,
    out_shape=[jax.ShapeDtypeStruct((NPAD, 1), jnp.float32),
               jax.ShapeDtypeStruct((NC, NPAD, HALF), jnp.float32)],
)

_tc_mid = pl.pallas_call(
    _tc_mid_body,
    out_shape=jax.ShapeDtypeStruct((NC, NPAD, HALF), jnp.float32),
)

_tc_post = pl.pallas_call(
    _tc_post_body,
    out_shape=jax.ShapeDtypeStruct((NPAD, D), jnp.float32),
)


def _fold_bn(W, b, g, be, m, v):
    s = g * lax.rsqrt(v + BN_EPS)
    return W * s[None, :], ((b - m) * s + be)[None, :]


def kernel(x, edge_index, edge_attr, W1, b1, g1, be1, m1, v1,
           W2, b2, g2, be2, m2, v2):
    row = edge_index[0]
    col = edge_index[1]
    zpad_i = jnp.zeros((EPAD - E,), jnp.int32)
    zpad_f = jnp.zeros((EPAD - E,), jnp.float32)
    rowp = jnp.concatenate([row, zpad_i])
    colp = jnp.concatenate([col, zpad_i])
    wp = jnp.concatenate([edge_attr, zpad_f])
    edata = jnp.stack([rowp.reshape(-1, CHUNK), colp.reshape(-1, CHUNK),
                       wp.view(jnp.int32).reshape(-1, CHUNK)], axis=1)
    xp = jnp.pad(x, ((0, NPAD - N), (0, 0)))
    W1e, b1e = _fold_bn(W1, b1, g1, be1, m1, v1)
    W2e, b2e = _fold_bn(W2, b2, g2, be2, m2, v2)

    deg = _deg_kernel(colp, wp)
    dis, xw1 = _tc_pre(deg, xp, W1e)
    acc1 = _agg_kernel(xw1, edata)
    xw2 = _tc_mid(acc1, dis, b1e, W2e)
    acc2 = _agg_kernel(xw2, edata)
    out = _tc_post(acc2, dis, b2e)
    return out[:N]
